# baseline (speedup 1.0000x reference)
import functools

import jax
import jax.numpy as jnp
from jax import lax
from jax.experimental import pallas as pl
from jax.experimental.pallas import tpu as pltpu
from jax.experimental.pallas import tpu_sc as plsc

B = 4096
D = 64
NEG = 5
NW = 32
K = 16
CHUNK = 128

N_U = B * (2 + NEG)
N_C = B * (1 + NEG)
UPW = N_U // NW
CPW = N_C // NW
UCH = UPW // CHUNK

_mesh = plsc.VectorSubcoreMesh(core_axis_name="c", subcore_axis_name="s")


@functools.partial(
    pl.kernel,
    out_type=jax.ShapeDtypeStruct((N_U, D), jnp.float32),
    mesh=_mesh,
    scratch_types=[
        pltpu.VMEM((UPW,), jnp.int32),
        pltpu.VMEM((UPW, D), jnp.float32),
        pltpu.SemaphoreType.DMA,
    ],
)
def _sc_gather_u(u_hbm, idx_hbm, out, idx_v, rows, sem):
    wid = lax.axis_index("s") * 2 + lax.axis_index("c")
    pltpu.sync_copy(idx_hbm.at[wid], idx_v)

    def chunk(c, _):
        base = c * K
        vec = idx_v[pl.ds(base, K)]
        cps = []
        for j in range(K):
            r = vec[j]
            cps.append(pltpu.async_copy(
                u_hbm.at[pl.ds(r, 1)], rows.at[pl.ds(base + j, 1)], sem))
        for cp in cps:
            cp.wait()
        return _

    lax.fori_loop(0, UPW // K, chunk, 0, unroll=False)
    pltpu.sync_copy(rows, out.at[pl.ds(wid * UPW, UPW)])


@functools.partial(
    pl.kernel,
    out_type=jax.ShapeDtypeStruct((N_C, D), jnp.float32),
    mesh=_mesh,
    scratch_types=[
        pltpu.VMEM((CPW,), jnp.int32),
        pltpu.VMEM((CPW, D), jnp.float32),
        pltpu.SemaphoreType.DMA,
    ],
)
def _sc_gather_c(c_hbm, idx_hbm, out, idx_v, rows, sem):
    wid = lax.axis_index("s") * 2 + lax.axis_index("c")
    pltpu.sync_copy(idx_hbm.at[wid], idx_v)

    def chunk(c, _):
        base = c * K
        vec = idx_v[pl.ds(base, K)]
        cps = []
        for j in range(K):
            r = vec[j]
            cps.append(pltpu.async_copy(
                c_hbm.at[pl.ds(r, 1)], rows.at[pl.ds(base + j, 1)], sem))
        for cp in cps:
            cp.wait()
        return _

    lax.fori_loop(0, CPW // K, chunk, 0, unroll=False)
    pltpu.sync_copy(rows, out.at[pl.ds(wid * CPW, CPW)])


_TNB = 32768
_TSTEPS = -(-1000000 // _TNB)


def _tr_body(in_ref, out_ref):
    out_ref[...] = in_ref[...].T


_tc_transpose = pl.pallas_call(
    _tr_body,
    grid=(_TSTEPS,),
    in_specs=[pl.BlockSpec((D, _TNB), lambda i: (0, i))],
    out_specs=pl.BlockSpec((_TNB, D), lambda i: (i, 0)),
    out_shape=jax.ShapeDtypeStruct((1000000, D), jnp.float32),
)


def _logsig(x):
    return jnp.minimum(x, 0.0) - jnp.log1p(jnp.exp(-jnp.abs(x)))


_IBLK = 512


def _tc_u_body(gu_ref, vot_u_ref, out_ref):
    vi = gu_ref[0:B, :]
    vib = vi.astype(jnp.bfloat16)
    neg_total = 0.0
    for k in range(NEG):
        ngk = gu_ref[2 * B + k * B: 2 * B + (k + 1) * B, :]
        d = jnp.sum(vi * ngk, axis=1)
        neg_total += jnp.sum(_logsig(-d))
    vot_u = vot_u_ref[...].astype(jnp.bfloat16)
    pos_total = 0.0
    for i in range(B // _IBLK):
        blk = vib[i * _IBLK:(i + 1) * _IBLK, :]
        l1 = jnp.dot(blk, vot_u, preferred_element_type=jnp.float32)
        pos_total += jnp.sum(_logsig(l1))
    out_ref[0, 0] = pos_total / (B * B) + neg_total / B


def _tc_c_body(gc_ref, gu_ref, vot_c_ref, part_ref, out_ref):
    vi = gu_ref[0:B, :]
    vib = vi.astype(jnp.bfloat16)
    neg_total = 0.0
    for k in range(NEG):
        ngk = gc_ref[B + k * B: B + (k + 1) * B, :]
        d = jnp.sum(vi * ngk, axis=1)
        neg_total += jnp.sum(_logsig(-d))
    vot_c = vot_c_ref[...].astype(jnp.bfloat16)
    pos_total = 0.0
    for i in range(B // _IBLK):
        blk = vib[i * _IBLK:(i + 1) * _IBLK, :]
        l2 = jnp.dot(blk, vot_c, preferred_element_type=jnp.float32)
        pos_total += jnp.sum(_logsig(l2))
    out_ref[0, 0] = -(part_ref[0, 0] + pos_total / (B * B) + neg_total / B)


_tc_reduce_u = pl.pallas_call(
    _tc_u_body,
    out_shape=jax.ShapeDtypeStruct((1, 1), jnp.float32),
    out_specs=pl.BlockSpec(memory_space=pltpu.SMEM),
)

_tc_reduce_c = pl.pallas_call(
    _tc_c_body,
    in_specs=[pl.BlockSpec(memory_space=pltpu.ANY).replace(memory_space=None)
              if False else pl.BlockSpec(),
              pl.BlockSpec(), pl.BlockSpec(),
              pl.BlockSpec(memory_space=pltpu.SMEM)],
    out_shape=jax.ShapeDtypeStruct((1, 1), jnp.float32),
    out_specs=pl.BlockSpec(memory_space=pltpu.SMEM),
)


def kernel(data, u_emd, context_emd):
    negs = data[:, 2:].T.reshape(-1)
    idx_u = jnp.concatenate([data[:, 0], data[:, 1], negs])
    idx_c = jnp.concatenate([data[:, 1], negs])
    us = _tc_transpose(jnp.swapaxes(u_emd, 0, 1))
    cs = _tc_transpose(jnp.swapaxes(context_emd, 0, 1))
    gu = _sc_gather_u(us, idx_u.reshape(NW, UPW))
    gc = _sc_gather_c(cs, idx_c.reshape(NW, CPW))
    vot_u = gu[B:2 * B, :].T
    vot_c = gc[0:B, :].T
    part = _tc_reduce_u(gu, vot_u)
    return _tc_reduce_c(gc, gu, vot_c, part)[0, 0]


# speedup vs baseline: 1.0338x; 1.0338x over previous
import functools

import jax
import jax.numpy as jnp
from jax import lax
from jax.experimental import pallas as pl
from jax.experimental.pallas import tpu as pltpu
from jax.experimental.pallas import tpu_sc as plsc

B = 4096
D = 64
NEG = 5
NW = 32
K = 32
CHUNK = 128

N_U = B * (2 + NEG)
N_C = B * (1 + NEG)
UPW = N_U // NW
CPW = N_C // NW
UCH = UPW // CHUNK

_mesh = plsc.VectorSubcoreMesh(core_axis_name="c", subcore_axis_name="s")


@functools.partial(
    pl.kernel,
    out_type=jax.ShapeDtypeStruct((N_U, D), jnp.float32),
    mesh=_mesh,
    scratch_types=[
        pltpu.VMEM((UPW,), jnp.int32),
        pltpu.VMEM((UPW, D), jnp.float32),
        pltpu.SemaphoreType.DMA,
    ],
)
def _sc_gather_u(u_hbm, idx_hbm, out, idx_v, rows, sem):
    wid = lax.axis_index("s") * 2 + lax.axis_index("c")
    pltpu.sync_copy(idx_hbm.at[wid], idx_v)

    def chunk(c, _):
        base = c * K
        vec = idx_v[pl.ds(base, K)]
        cps = []
        for j in range(K):
            r = vec[j]
            cps.append(pltpu.async_copy(
                u_hbm.at[pl.ds(r, 1)], rows.at[pl.ds(base + j, 1)], sem))
        for cp in cps:
            cp.wait()
        return _

    lax.fori_loop(0, UPW // K, chunk, 0, unroll=False)
    pltpu.sync_copy(rows, out.at[pl.ds(wid * UPW, UPW)])


@functools.partial(
    pl.kernel,
    out_type=jax.ShapeDtypeStruct((N_C, D), jnp.float32),
    mesh=_mesh,
    scratch_types=[
        pltpu.VMEM((CPW,), jnp.int32),
        pltpu.VMEM((CPW, D), jnp.float32),
        pltpu.SemaphoreType.DMA,
    ],
)
def _sc_gather_c(c_hbm, idx_hbm, out, idx_v, rows, sem):
    wid = lax.axis_index("s") * 2 + lax.axis_index("c")
    pltpu.sync_copy(idx_hbm.at[wid], idx_v)

    def chunk(c, _):
        base = c * K
        vec = idx_v[pl.ds(base, K)]
        cps = []
        for j in range(K):
            r = vec[j]
            cps.append(pltpu.async_copy(
                c_hbm.at[pl.ds(r, 1)], rows.at[pl.ds(base + j, 1)], sem))
        for cp in cps:
            cp.wait()
        return _

    lax.fori_loop(0, CPW // K, chunk, 0, unroll=False)
    pltpu.sync_copy(rows, out.at[pl.ds(wid * CPW, CPW)])


_TNB = 32768
_TSTEPS = -(-1000000 // _TNB)


def _tr_body(in_ref, out_ref):
    out_ref[...] = in_ref[...].T


_tc_transpose = pl.pallas_call(
    _tr_body,
    grid=(_TSTEPS,),
    in_specs=[pl.BlockSpec((D, _TNB), lambda i: (0, i))],
    out_specs=pl.BlockSpec((_TNB, D), lambda i: (i, 0)),
    out_shape=jax.ShapeDtypeStruct((1000000, D), jnp.float32),
)


def _logsig(x):
    return jnp.minimum(x, 0.0) - jnp.log1p(jnp.exp(-jnp.abs(x)))


_IBLK = 512


def _tc_body(gu_ref, gc_ref, vot_u_ref, vot_c_ref, out_ref):
    vi = gu_ref[0:B, :]
    vib = vi.astype(jnp.bfloat16)
    neg_total = 0.0
    for ref, off in ((gu_ref, 2 * B), (gc_ref, B)):
        for k in range(NEG):
            ngk = ref[off + k * B: off + (k + 1) * B, :]
            d = jnp.sum(vi * ngk, axis=1)
            neg_total += jnp.sum(_logsig(-d))
    vot_u = vot_u_ref[...].astype(jnp.bfloat16)
    vot_c = vot_c_ref[...].astype(jnp.bfloat16)
    pos_total = 0.0
    for i in range(B // _IBLK):
        blk = vib[i * _IBLK:(i + 1) * _IBLK, :]
        l1 = jnp.dot(blk, vot_u, preferred_element_type=jnp.float32)
        l2 = jnp.dot(blk, vot_c, preferred_element_type=jnp.float32)
        pos_total += jnp.sum(_logsig(l1)) + jnp.sum(_logsig(l2))
    out_ref[0, 0] = -(pos_total / (B * B) + neg_total / B)


_tc_reduce = pl.pallas_call(
    _tc_body,
    out_shape=jax.ShapeDtypeStruct((1, 1), jnp.float32),
    out_specs=pl.BlockSpec(memory_space=pltpu.SMEM),
)


def kernel(data, u_emd, context_emd):
    negs = data[:, 2:].T.reshape(-1)
    idx_u = jnp.concatenate([data[:, 0], data[:, 1], negs])
    idx_c = jnp.concatenate([data[:, 1], negs])
    us = _tc_transpose(jnp.swapaxes(u_emd, 0, 1))
    cs = _tc_transpose(jnp.swapaxes(context_emd, 0, 1))
    gu = _sc_gather_u(us, idx_u.reshape(NW, UPW))
    gc = _sc_gather_c(cs, idx_c.reshape(NW, CPW))
    vot_u = gu[B:2 * B, :].T
    vot_c = gc[0:B, :].T
    return _tc_reduce(gu, gc, vot_u, vot_c)[0, 0]


# speedup vs baseline: 1.0480x; 1.0137x over previous
import functools

import jax
import jax.numpy as jnp
from jax import lax
from jax.experimental import pallas as pl
from jax.experimental.pallas import tpu as pltpu
from jax.experimental.pallas import tpu_sc as plsc

B = 4096
D = 64
NEG = 5
NW = 32
K = 64
CHUNK = 128

N_U = B * (2 + NEG)
N_C = B * (1 + NEG)
UPW = N_U // NW
CPW = N_C // NW
UCH = UPW // CHUNK

_mesh = plsc.VectorSubcoreMesh(core_axis_name="c", subcore_axis_name="s")


@functools.partial(
    pl.kernel,
    out_type=jax.ShapeDtypeStruct((N_U, D), jnp.float32),
    mesh=_mesh,
    scratch_types=[
        pltpu.VMEM((UPW,), jnp.int32),
        pltpu.VMEM((UPW, D), jnp.float32),
        pltpu.SemaphoreType.DMA,
    ],
)
def _sc_gather_u(u_hbm, idx_hbm, out, idx_v, rows, sem):
    wid = lax.axis_index("s") * 2 + lax.axis_index("c")
    pltpu.sync_copy(idx_hbm.at[wid], idx_v)

    def chunk(c, _):
        base = c * K
        vec = idx_v[pl.ds(base, K)]
        cps = []
        for j in range(K):
            r = vec[j]
            cps.append(pltpu.async_copy(
                u_hbm.at[pl.ds(r, 1)], rows.at[pl.ds(base + j, 1)], sem))
        for cp in cps:
            cp.wait()
        return _

    lax.fori_loop(0, UPW // K, chunk, 0, unroll=False)
    pltpu.sync_copy(rows, out.at[pl.ds(wid * UPW, UPW)])


@functools.partial(
    pl.kernel,
    out_type=jax.ShapeDtypeStruct((N_C, D), jnp.float32),
    mesh=_mesh,
    scratch_types=[
        pltpu.VMEM((CPW,), jnp.int32),
        pltpu.VMEM((CPW, D), jnp.float32),
        pltpu.SemaphoreType.DMA,
    ],
)
def _sc_gather_c(c_hbm, idx_hbm, out, idx_v, rows, sem):
    wid = lax.axis_index("s") * 2 + lax.axis_index("c")
    pltpu.sync_copy(idx_hbm.at[wid], idx_v)

    def chunk(c, _):
        base = c * K
        vec = idx_v[pl.ds(base, K)]
        cps = []
        for j in range(K):
            r = vec[j]
            cps.append(pltpu.async_copy(
                c_hbm.at[pl.ds(r, 1)], rows.at[pl.ds(base + j, 1)], sem))
        for cp in cps:
            cp.wait()
        return _

    lax.fori_loop(0, CPW // K, chunk, 0, unroll=False)
    pltpu.sync_copy(rows, out.at[pl.ds(wid * CPW, CPW)])


_TNB = 32768
_TSTEPS = -(-1000000 // _TNB)


def _tr_body(in_ref, out_ref):
    out_ref[...] = in_ref[...].T


_tc_transpose = pl.pallas_call(
    _tr_body,
    grid=(_TSTEPS,),
    in_specs=[pl.BlockSpec((D, _TNB), lambda i: (0, i))],
    out_specs=pl.BlockSpec((_TNB, D), lambda i: (i, 0)),
    out_shape=jax.ShapeDtypeStruct((1000000, D), jnp.float32),
)


def _logsig(x):
    return jnp.minimum(x, 0.0) - jnp.log1p(jnp.exp(-jnp.abs(x)))


_IBLK = 512


def _tc_body(gu_ref, gc_ref, vot_u_ref, vot_c_ref, out_ref):
    vi = gu_ref[0:B, :]
    vib = vi.astype(jnp.bfloat16)
    neg_total = 0.0
    for ref, off in ((gu_ref, 2 * B), (gc_ref, B)):
        for k in range(NEG):
            ngk = ref[off + k * B: off + (k + 1) * B, :]
            d = jnp.sum(vi * ngk, axis=1)
            neg_total += jnp.sum(_logsig(-d))
    vot_u = vot_u_ref[...].astype(jnp.bfloat16)
    vot_c = vot_c_ref[...].astype(jnp.bfloat16)
    pos_total = 0.0
    for i in range(B // _IBLK):
        blk = vib[i * _IBLK:(i + 1) * _IBLK, :]
        l1 = jnp.dot(blk, vot_u, preferred_element_type=jnp.float32)
        l2 = jnp.dot(blk, vot_c, preferred_element_type=jnp.float32)
        pos_total += jnp.sum(_logsig(l1)) + jnp.sum(_logsig(l2))
    out_ref[0, 0] = -(pos_total / (B * B) + neg_total / B)


_tc_reduce = pl.pallas_call(
    _tc_body,
    out_shape=jax.ShapeDtypeStruct((1, 1), jnp.float32),
    out_specs=pl.BlockSpec(memory_space=pltpu.SMEM),
)


def kernel(data, u_emd, context_emd):
    negs = data[:, 2:].T.reshape(-1)
    idx_u = jnp.concatenate([data[:, 0], data[:, 1], negs])
    idx_c = jnp.concatenate([data[:, 1], negs])
    us = _tc_transpose(jnp.swapaxes(u_emd, 0, 1))
    cs = _tc_transpose(jnp.swapaxes(context_emd, 0, 1))
    gu = _sc_gather_u(us, idx_u.reshape(NW, UPW))
    gc = _sc_gather_c(cs, idx_c.reshape(NW, CPW))
    vot_u = gu[B:2 * B, :].T
    vot_c = gc[0:B, :].T
    return _tc_reduce(gu, gc, vot_u, vot_c)[0, 0]
